# baseline (speedup 1.0000x reference)
import functools

import jax
import jax.numpy as jnp
from jax import lax
from jax.experimental import pallas as pl
from jax.experimental.pallas import tpu as pltpu
from jax.experimental.pallas import tpu_sc as plsc

N_NODES = 10000
N_EDGES = 320000
D_FEAT = 128
D_EDGE = 16
OUT_DIM = 128

CHUNK = 32
N_CHUNKS = N_EDGES // CHUNK
NC, NS = 2, 16
NW = NC * NS
CPT = N_CHUNKS // NW
NBUF = 2
SPC = 8
CNT_ROWS = 80
ACC_ROWS = 10400
ROWS_PER_SUB = 624


def _tbl_body(n_ref, w0s_ref, w0r_ref, t_ref):
    i = pl.program_id(0)

    @pl.when(i < 10)
    def _a():
        t_ref[...] = jnp.dot(n_ref[...], w0s_ref[...],
                             preferred_element_type=jnp.float32)

    @pl.when(jnp.logical_and(i >= 10, i < 20))
    def _b():
        t_ref[...] = jnp.dot(n_ref[...], w0r_ref[...],
                             preferred_element_type=jnp.float32)

    @pl.when(i == 20)
    def _eye():
        r_id = lax.broadcasted_iota(jnp.int32, t_ref.shape, 0)
        c_id = lax.broadcasted_iota(jnp.int32, t_ref.shape, 1)
        t_ref[...] = (r_id == c_id).astype(jnp.float32)


def _c_body(e_ref, w0e_ref, b0_ref, c_ref):
    c_ref[...] = (
        jnp.dot(e_ref[...], w0e_ref[...], preferred_element_type=jnp.float32)
        + b0_ref[...]
    )


def _sc_body(senders, receivers, tbl_hbm, c_hbm, s_out,
             sidx, ridx, gidx, cidx, gbuf, dbuf,
             s_acc, sem_g, sem_c, sem_s):
    core = lax.axis_index("c")
    sub = lax.axis_index("s")
    wid = sub * NC + core

    zeros16 = jnp.zeros((16,), jnp.float32)

    def _fill_z(i, _):
        for j in range(D_FEAT // 16):
            gbuf[0][i, pl.ds(j * 16, 16)] = zeros16
        return 0
    lax.fori_loop(0, CHUNK, _fill_z, 0)

    zbase = sub * ROWS_PER_SUB
    nfull = ROWS_PER_SUB // CHUNK
    zrem = ROWS_PER_SUB - nfull * CHUNK
    zsrc = gbuf[0].at[pl.ds(0, CHUNK)]
    for k in range(nfull):
        pltpu.sync_copy(zsrc, s_acc.at[pl.ds(zbase + k * CHUNK, CHUNK)])
    pltpu.sync_copy(gbuf[0].at[pl.ds(0, zrem)],
                    s_acc.at[pl.ds(zbase + nfull * CHUNK, zrem)])

    @pl.when(sub == NS - 1)
    def _zero_tail():
        tail = NS * ROWS_PER_SUB
        for k in range((ACC_ROWS - tail) // CHUNK):
            pltpu.sync_copy(zsrc, s_acc.at[pl.ds(tail + k * CHUNK, CHUNK)])

    plsc.subcore_barrier()

    NSC = CPT // SPC
    q_lo = wid * CPT
    r_lo = q_lo

    def _load_idx(sk, s):
        pltpu.sync_copy(senders.at[pl.ds(r_lo + sk * SPC, SPC)], sidx[s])
        pltpu.sync_copy(receivers.at[pl.ds(r_lo + sk * SPC, SPC)], ridx[s])

    def _build_idx(s, j, b):
        for g in range(CHUNK // 16):
            sl = pl.ds(g * 16, 16)
            sv = sidx[s][j, sl]
            rv = ridx[s][j, sl]
            gidx[b][sl] = sv
            gidx[b][pl.ds(CHUNK + g * 16, 16)] = rv + N_NODES
            gidx[b][pl.ds(2 * CHUNK + g * 16, 16)] = (
                lax.bitwise_and(sv, 127) + 2 * N_NODES)
            cidx[b][sl] = sv
            cidx[b][pl.ds(CHUNK + g * 16, 16)] = (
                lax.shift_right_logical(sv, 7) + N_NODES)

    def _fire(sk, j, s, b):
        base = (q_lo + sk * SPC + j) * CHUNK
        _build_idx(s, j, b)
        pltpu.async_copy(tbl_hbm.at[gidx[b]], gbuf[b], sem_g[b])
        pltpu.async_copy(c_hbm.at[pl.ds(base, CHUNK)], dbuf[b], sem_c[b])

    def _compute(b):
        def _row(r, _):
            for jj in range(D_FEAT // 16):
                sl = pl.ds(jj * 16, 16)
                gbuf[b][CHUNK + r, sl] = jnp.maximum(
                    gbuf[b][r, sl] + gbuf[b][CHUNK + r, sl]
                    + dbuf[b][r, sl], 0.0)
            return 0
        lax.fori_loop(0, CHUNK, _row, 0)

    _load_idx(0, 0)
    for b in range(NBUF):
        _fire(0, b, 0, b)

    def _one_super(sk, s, last):
        if not last:
            _load_idx(sk + 1, 1 - s)

        for j in range(SPC):
            b = j % NBUF
            pltpu.make_async_copy(tbl_hbm.at[gidx[b]], gbuf[b],
                                  sem_g[b]).wait()
            pltpu.make_async_copy(c_hbm.at[pl.ds(0, CHUNK)], dbuf[b],
                                  sem_c[b]).wait()
            _compute(b)
            pltpu.async_copy(gbuf[b].at[pl.ds(CHUNK, 2 * CHUNK)],
                             s_acc.at[cidx[b]], sem_s[b],
                             add=True).wait()
            if j + NBUF < SPC:
                _fire(sk, j + NBUF, s, b)
            elif not last:
                _fire(sk + 1, j + NBUF - SPC, 1 - s, b)

    def _pair(p, _):
        _one_super(2 * p, 0, False)
        _one_super(2 * p + 1, 1, False)
        return 0

    lax.fori_loop(0, (NSC - 1) // 2, _pair, 0)
    _one_super(NSC - 1, 0, True)

    @pl.when(wid < N_CHUNKS - NW * CPT)
    def _leftover():
        qabs = NW * CPT + wid
        pltpu.sync_copy(senders.at[pl.ds(qabs, 1)], sidx[0].at[pl.ds(0, 1)])
        pltpu.sync_copy(receivers.at[pl.ds(qabs, 1)], ridx[0].at[pl.ds(0, 1)])
        _build_idx(0, 0, 0)
        cp_g = pltpu.async_copy(tbl_hbm.at[gidx[0]], gbuf[0], sem_g[0])
        cp_c = pltpu.async_copy(c_hbm.at[pl.ds(qabs * CHUNK, CHUNK)], dbuf[0],
                                sem_c[0])
        cp_g.wait()
        cp_c.wait()
        _compute(0)
        pltpu.async_copy(gbuf[0].at[pl.ds(CHUNK, 2 * CHUNK)],
                         s_acc.at[cidx[0]], sem_s[0], add=True).wait()

    plsc.subcore_barrier()

    obase = core * ACC_ROWS + zbase
    stg = gbuf[0].at[pl.ds(0, CHUNK)]
    for k in range(nfull):
        pltpu.sync_copy(s_acc.at[pl.ds(zbase + k * CHUNK, CHUNK)], stg)
        pltpu.sync_copy(stg, s_out.at[pl.ds(obase + k * CHUNK, CHUNK)])
    pltpu.sync_copy(s_acc.at[pl.ds(zbase + nfull * CHUNK, zrem)],
                    gbuf[0].at[pl.ds(0, zrem)])
    pltpu.sync_copy(gbuf[0].at[pl.ds(0, zrem)],
                    s_out.at[pl.ds(obase + nfull * CHUNK, zrem)])

    @pl.when(sub == NS - 1)
    def _write_tail():
        tail = NS * ROWS_PER_SUB
        for k in range((ACC_ROWS - tail) // CHUNK):
            pltpu.sync_copy(s_acc.at[pl.ds(tail + k * CHUNK, CHUNK)], stg)
            pltpu.sync_copy(
                stg,
                s_out.at[pl.ds(core * ACC_ROWS + tail + k * CHUNK, CHUNK)])


def _out_body(s0_ref, s1_ref, c0_ref, c1_ref, w1_ref, b1_ref, o_ref):
    s = s0_ref[...] + s1_ref[...]
    cnt = c0_ref[...] + c1_ref[...]
    m = s / jnp.maximum(cnt, 1.0)
    o_ref[...] = (
        jnp.dot(m, w1_ref[...], preferred_element_type=jnp.float32)
        + jnp.where(cnt > 0.0, b1_ref[...], 0.0)
    )


def kernel(n_embed, e_embed, senders, receivers, W0, b0, W1, b1):
    w0s = W0[:D_FEAT]
    w0r = W0[D_FEAT:2 * D_FEAT]
    w0e = W0[2 * D_FEAT:]
    b0r = b0.reshape(1, OUT_DIM)
    b1r = b1.reshape(1, OUT_DIM)

    nb = 1000
    tbl = pl.pallas_call(
        _tbl_body,
        grid=(21,),
        in_specs=[
            pl.BlockSpec((nb, D_FEAT), lambda i: (lax.rem(i, 10), 0)),
            pl.BlockSpec((D_FEAT, OUT_DIM), lambda i: (0, 0)),
            pl.BlockSpec((D_FEAT, OUT_DIM), lambda i: (0, 0)),
        ],
        out_specs=pl.BlockSpec((nb, OUT_DIM), lambda i: (i, 0)),
        out_shape=jax.ShapeDtypeStruct((21 * nb, OUT_DIM), jnp.float32),
    )(n_embed, w0s, w0r)

    eb = 4000
    c_mat = pl.pallas_call(
        _c_body,
        grid=(N_EDGES // eb,),
        in_specs=[
            pl.BlockSpec((eb, D_EDGE), lambda i: (i, 0)),
            pl.BlockSpec((D_EDGE, OUT_DIM), lambda i: (0, 0)),
            pl.BlockSpec((1, OUT_DIM), lambda i: (0, 0)),
        ],
        out_specs=pl.BlockSpec((eb, OUT_DIM), lambda i: (i, 0)),
        out_shape=jax.ShapeDtypeStruct((N_EDGES, OUT_DIM), jnp.float32),
    )(e_embed, w0e, b0r)

    sc_fn = pl.kernel(
        _sc_body,
        out_type=jax.ShapeDtypeStruct((NC * ACC_ROWS, OUT_DIM), jnp.float32),
        mesh=plsc.VectorSubcoreMesh(core_axis_name="c", subcore_axis_name="s"),
        scratch_types=[
            [pltpu.VMEM((SPC, CHUNK), jnp.int32)] * 2,
            [pltpu.VMEM((SPC, CHUNK), jnp.int32)] * 2,
            [pltpu.VMEM((3 * CHUNK,), jnp.int32)] * NBUF,
            [pltpu.VMEM((2 * CHUNK,), jnp.int32)] * NBUF,
            [pltpu.VMEM((3 * CHUNK, D_FEAT), jnp.float32)] * NBUF,
            [pltpu.VMEM((CHUNK, D_FEAT), jnp.float32)] * NBUF,
            pltpu.VMEM_SHARED((ACC_ROWS, OUT_DIM), jnp.float32),
            [pltpu.SemaphoreType.DMA] * NBUF,
            [pltpu.SemaphoreType.DMA] * NBUF,
            [pltpu.SemaphoreType.DMA] * NBUF,
        ],
    )
    s2 = senders.reshape(N_CHUNKS, CHUNK)
    r2 = receivers.reshape(N_CHUNKS, CHUNK)
    s_part = sc_fn(s2, r2, tbl, c_mat)

    cnt0 = (s_part[N_NODES:N_NODES + CNT_ROWS]
            .reshape(-1)[:N_NODES].reshape(N_NODES, 1))
    cnt1 = (s_part[ACC_ROWS + N_NODES:ACC_ROWS + N_NODES + CNT_ROWS]
            .reshape(-1)[:N_NODES].reshape(N_NODES, 1))

    ob = 400
    out = pl.pallas_call(
        _out_body,
        grid=(N_NODES // ob,),
        in_specs=[
            pl.BlockSpec((ob, OUT_DIM), lambda i: (i, 0)),
            pl.BlockSpec((ob, OUT_DIM), lambda i: (i + ACC_ROWS // ob, 0)),
            pl.BlockSpec((ob, 1), lambda i: (i, 0)),
            pl.BlockSpec((ob, 1), lambda i: (i, 0)),
            pl.BlockSpec((OUT_DIM, OUT_DIM), lambda i: (0, 0)),
            pl.BlockSpec((1, OUT_DIM), lambda i: (0, 0)),
        ],
        out_specs=pl.BlockSpec((ob, OUT_DIM), lambda i: (i, 0)),
        out_shape=jax.ShapeDtypeStruct((N_NODES, OUT_DIM), jnp.float32),
    )(s_part, s_part, cnt0, cnt1, W1, b1r)
    return out


# speedup vs baseline: 1.0202x; 1.0202x over previous
import functools

import jax
import jax.numpy as jnp
from jax import lax
from jax.experimental import pallas as pl
from jax.experimental.pallas import tpu as pltpu
from jax.experimental.pallas import tpu_sc as plsc

N_NODES = 10000
N_EDGES = 320000
D_FEAT = 128
D_EDGE = 16
OUT_DIM = 128

CHUNK = 32
N_CHUNKS = N_EDGES // CHUNK
NC, NS = 2, 16
NW = NC * NS
CPT = N_CHUNKS // NW
NBUF = 2
SPC = 8
CNT_ROWS = 80
ACC_ROWS = N_NODES + CNT_ROWS
ROWS_PER_SUB = 624


def _tbl_body(n_ref, w0s_ref, w0r_ref, t_ref):
    i = pl.program_id(0)
    x = n_ref[...]

    @pl.when(i < 10)
    def _a():
        t_ref[...] = jnp.dot(x, w0s_ref[...],
                             preferred_element_type=jnp.float32)

    @pl.when(jnp.logical_and(i >= 10, i < 20))
    def _b():
        t_ref[...] = jnp.dot(x, w0r_ref[...],
                             preferred_element_type=jnp.float32)

    @pl.when(i == 20)
    def _eye():
        r_id = lax.broadcasted_iota(jnp.int32, t_ref.shape, 0)
        c_id = lax.broadcasted_iota(jnp.int32, t_ref.shape, 1)
        t_ref[...] = (r_id == c_id).astype(jnp.float32)


def _c_body(e_ref, w0e_ref, b0_ref, c_ref):
    c_ref[...] = (
        jnp.dot(e_ref[...], w0e_ref[...], preferred_element_type=jnp.float32)
        + b0_ref[...]
    )


def _sc_body(senders, receivers, tbl_hbm, c_hbm, s_out,
             sidx, ridx, gidx, cidx, gbuf, dbuf,
             s_acc, sem_g, sem_c, sem_s):
    core = lax.axis_index("c")
    sub = lax.axis_index("s")
    wid = sub * NC + core

    zeros16 = jnp.zeros((16,), jnp.float32)

    def _fill_z(i, _):
        for j in range(D_FEAT // 16):
            gbuf[0][i, pl.ds(j * 16, 16)] = zeros16
        return 0
    lax.fori_loop(0, CHUNK, _fill_z, 0)

    zbase = sub * ROWS_PER_SUB
    nfull = ROWS_PER_SUB // CHUNK
    zrem = ROWS_PER_SUB - nfull * CHUNK
    zsrc = gbuf[0].at[pl.ds(0, CHUNK)]
    for k in range(nfull):
        pltpu.sync_copy(zsrc, s_acc.at[pl.ds(zbase + k * CHUNK, CHUNK)])
    pltpu.sync_copy(gbuf[0].at[pl.ds(0, zrem)],
                    s_acc.at[pl.ds(zbase + nfull * CHUNK, zrem)])

    @pl.when(sub == NS - 1)
    def _zero_tail():
        tail = NS * ROWS_PER_SUB
        for k in range((ACC_ROWS - tail) // CHUNK):
            pltpu.sync_copy(zsrc, s_acc.at[pl.ds(tail + k * CHUNK, CHUNK)])

    plsc.subcore_barrier()

    NSC = CPT // SPC
    q_lo = wid * CPT
    r_lo = q_lo

    def _load_idx(sk, s):
        pltpu.sync_copy(senders.at[pl.ds(r_lo + sk * SPC, SPC)], sidx[s])
        pltpu.sync_copy(receivers.at[pl.ds(r_lo + sk * SPC, SPC)], ridx[s])

    def _build_idx(s, j, b):
        for g in range(CHUNK // 16):
            sl = pl.ds(g * 16, 16)
            sv = sidx[s][j, sl]
            rv = ridx[s][j, sl]
            gidx[b][sl] = sv
            gidx[b][pl.ds(CHUNK + g * 16, 16)] = rv + N_NODES
            gidx[b][pl.ds(2 * CHUNK + g * 16, 16)] = (
                lax.bitwise_and(sv, 127) + 2 * N_NODES)
            cidx[b][sl] = sv
            cidx[b][pl.ds(CHUNK + g * 16, 16)] = (
                lax.shift_right_logical(sv, 7) + N_NODES)

    def _fire(sk, j, s, b):
        base = (q_lo + sk * SPC + j) * CHUNK
        _build_idx(s, j, b)
        pltpu.async_copy(tbl_hbm.at[gidx[b]], gbuf[b], sem_g[b])
        pltpu.async_copy(c_hbm.at[pl.ds(base, CHUNK)], dbuf[b], sem_c[b])

    def _compute(b):
        def _row(r, _):
            for jj in range(D_FEAT // 16):
                sl = pl.ds(jj * 16, 16)
                gbuf[b][CHUNK + r, sl] = jnp.maximum(
                    gbuf[b][r, sl] + gbuf[b][CHUNK + r, sl]
                    + dbuf[b][r, sl], 0.0)
            return 0
        lax.fori_loop(0, CHUNK, _row, 0)

    _load_idx(0, 0)
    for b in range(NBUF):
        _fire(0, b, 0, b)

    def _one_super(sk, s, last):
        if not last:
            _load_idx(sk + 1, 1 - s)

        for j in range(SPC):
            b = j % NBUF
            pltpu.make_async_copy(tbl_hbm.at[gidx[b]], gbuf[b],
                                  sem_g[b]).wait()
            pltpu.make_async_copy(c_hbm.at[pl.ds(0, CHUNK)], dbuf[b],
                                  sem_c[b]).wait()
            _compute(b)
            pltpu.async_copy(gbuf[b].at[pl.ds(CHUNK, 2 * CHUNK)],
                             s_acc.at[cidx[b]], sem_s[b],
                             add=True).wait()
            if j + NBUF < SPC:
                _fire(sk, j + NBUF, s, b)
            elif not last:
                _fire(sk + 1, j + NBUF - SPC, 1 - s, b)

    def _pair(p, _):
        _one_super(2 * p, 0, False)
        _one_super(2 * p + 1, 1, False)
        return 0

    lax.fori_loop(0, (NSC - 1) // 2, _pair, 0)
    _one_super(NSC - 1, 0, True)

    @pl.when(wid < N_CHUNKS - NW * CPT)
    def _leftover():
        qabs = NW * CPT + wid
        pltpu.sync_copy(senders.at[pl.ds(qabs, 1)], sidx[0].at[pl.ds(0, 1)])
        pltpu.sync_copy(receivers.at[pl.ds(qabs, 1)], ridx[0].at[pl.ds(0, 1)])
        _build_idx(0, 0, 0)
        cp_g = pltpu.async_copy(tbl_hbm.at[gidx[0]], gbuf[0], sem_g[0])
        cp_c = pltpu.async_copy(c_hbm.at[pl.ds(qabs * CHUNK, CHUNK)], dbuf[0],
                                sem_c[0])
        cp_g.wait()
        cp_c.wait()
        _compute(0)
        pltpu.async_copy(gbuf[0].at[pl.ds(CHUNK, 2 * CHUNK)],
                         s_acc.at[cidx[0]], sem_s[0], add=True).wait()

    plsc.subcore_barrier()

    obase = core * ACC_ROWS + zbase
    stg = gbuf[0].at[pl.ds(0, CHUNK)]
    for k in range(nfull):
        pltpu.sync_copy(s_acc.at[pl.ds(zbase + k * CHUNK, CHUNK)], stg)
        pltpu.sync_copy(stg, s_out.at[pl.ds(obase + k * CHUNK, CHUNK)])
    pltpu.sync_copy(s_acc.at[pl.ds(zbase + nfull * CHUNK, zrem)],
                    gbuf[0].at[pl.ds(0, zrem)])
    pltpu.sync_copy(gbuf[0].at[pl.ds(0, zrem)],
                    s_out.at[pl.ds(obase + nfull * CHUNK, zrem)])

    @pl.when(sub == NS - 1)
    def _write_tail():
        tail = NS * ROWS_PER_SUB
        for k in range((ACC_ROWS - tail) // CHUNK):
            pltpu.sync_copy(s_acc.at[pl.ds(tail + k * CHUNK, CHUNK)], stg)
            pltpu.sync_copy(
                stg,
                s_out.at[pl.ds(core * ACC_ROWS + tail + k * CHUNK, CHUNK)])


def _out_body(s0_ref, s1_ref, c0_ref, c1_ref, w1_ref, b1_ref, o_ref):
    s = s0_ref[...] + s1_ref[...]
    cnt = c0_ref[...] + c1_ref[...]
    m = s / jnp.maximum(cnt, 1.0)
    o_ref[...] = (
        jnp.dot(m, w1_ref[...], preferred_element_type=jnp.float32)
        + jnp.where(cnt > 0.0, b1_ref[...], 0.0)
    )


def kernel(n_embed, e_embed, senders, receivers, W0, b0, W1, b1):
    w0s = W0[:D_FEAT]
    w0r = W0[D_FEAT:2 * D_FEAT]
    w0e = W0[2 * D_FEAT:]
    b0r = b0.reshape(1, OUT_DIM)
    b1r = b1.reshape(1, OUT_DIM)

    nb = 1000
    tbl = pl.pallas_call(
        _tbl_body,
        grid=(21,),
        in_specs=[
            pl.BlockSpec((nb, D_FEAT), lambda i: (lax.rem(i, 10), 0)),
            pl.BlockSpec((D_FEAT, OUT_DIM), lambda i: (0, 0)),
            pl.BlockSpec((D_FEAT, OUT_DIM), lambda i: (0, 0)),
        ],
        out_specs=pl.BlockSpec((nb, OUT_DIM), lambda i: (i, 0)),
        out_shape=jax.ShapeDtypeStruct((21 * nb, OUT_DIM), jnp.float32),
    )(n_embed, w0s, w0r)

    eb = 4000
    c_mat = pl.pallas_call(
        _c_body,
        grid=(N_EDGES // eb,),
        in_specs=[
            pl.BlockSpec((eb, D_EDGE), lambda i: (i, 0)),
            pl.BlockSpec((D_EDGE, OUT_DIM), lambda i: (0, 0)),
            pl.BlockSpec((1, OUT_DIM), lambda i: (0, 0)),
        ],
        out_specs=pl.BlockSpec((eb, OUT_DIM), lambda i: (i, 0)),
        out_shape=jax.ShapeDtypeStruct((N_EDGES, OUT_DIM), jnp.float32),
    )(e_embed, w0e, b0r)

    sc_fn = pl.kernel(
        _sc_body,
        out_type=jax.ShapeDtypeStruct((NC * ACC_ROWS, OUT_DIM), jnp.float32),
        mesh=plsc.VectorSubcoreMesh(core_axis_name="c", subcore_axis_name="s"),
        scratch_types=[
            [pltpu.VMEM((SPC, CHUNK), jnp.int32)] * 2,
            [pltpu.VMEM((SPC, CHUNK), jnp.int32)] * 2,
            [pltpu.VMEM((3 * CHUNK,), jnp.int32)] * NBUF,
            [pltpu.VMEM((2 * CHUNK,), jnp.int32)] * NBUF,
            [pltpu.VMEM((3 * CHUNK, D_FEAT), jnp.float32)] * NBUF,
            [pltpu.VMEM((CHUNK, D_FEAT), jnp.float32)] * NBUF,
            pltpu.VMEM_SHARED((ACC_ROWS, OUT_DIM), jnp.float32),
            [pltpu.SemaphoreType.DMA] * NBUF,
            [pltpu.SemaphoreType.DMA] * NBUF,
            [pltpu.SemaphoreType.DMA] * NBUF,
        ],
    )
    s2 = senders.reshape(N_CHUNKS, CHUNK)
    r2 = receivers.reshape(N_CHUNKS, CHUNK)
    s_part = sc_fn(s2, r2, tbl, c_mat)

    s0 = s_part[:N_NODES]
    s1 = s_part[ACC_ROWS:ACC_ROWS + N_NODES]
    cnt0 = (s_part[N_NODES:N_NODES + CNT_ROWS]
            .reshape(-1)[:N_NODES].reshape(N_NODES, 1))
    cnt1 = (s_part[ACC_ROWS + N_NODES:ACC_ROWS + N_NODES + CNT_ROWS]
            .reshape(-1)[:N_NODES].reshape(N_NODES, 1))

    ob = 1000
    out = pl.pallas_call(
        _out_body,
        grid=(N_NODES // ob,),
        in_specs=[
            pl.BlockSpec((ob, OUT_DIM), lambda i: (i, 0)),
            pl.BlockSpec((ob, OUT_DIM), lambda i: (i, 0)),
            pl.BlockSpec((ob, 1), lambda i: (i, 0)),
            pl.BlockSpec((ob, 1), lambda i: (i, 0)),
            pl.BlockSpec((OUT_DIM, OUT_DIM), lambda i: (0, 0)),
            pl.BlockSpec((1, OUT_DIM), lambda i: (0, 0)),
        ],
        out_specs=pl.BlockSpec((ob, OUT_DIM), lambda i: (i, 0)),
        out_shape=jax.ShapeDtypeStruct((N_NODES, OUT_DIM), jnp.float32),
    )(s0, s1, cnt0, cnt1, W1, b1r)
    return out
